# baseline (speedup 1.0000x reference)
import jax
import jax.numpy as jnp
from jax import lax
from jax.experimental import pallas as pl
from jax.experimental.pallas import tpu as pltpu
from jax.experimental.pallas import tpu_sc as plsc

_L = 16


def _oracle_body(lu_hbm, out_hbm, lu_v, out_v):
    cid = lax.axis_index("c")
    sid = lax.axis_index("s")

    @pl.when(jnp.logical_and(cid == 0, sid == 0))
    def _():
        pltpu.sync_copy(lu_hbm, lu_v.at[pl.ds(0, 4)])
        iota = lax.iota(jnp.int32, _L)
        zeros = jnp.zeros((_L,), jnp.int32)
        y_tl = plsc.load_gather(lu_v, [zeros])
        sel = jnp.where(y_tl <= 3, 1, jnp.where(y_tl <= 6, 2, 3))
        fo = plsc.load_gather(lu_v, [sel])
        out_v[...] = jnp.where(iota == fo, 1.0, 0.0).astype(jnp.float32)
        pltpu.sync_copy(out_v, out_hbm)


def kernel(input_ids, look_up):
    del input_ids
    lu = look_up.astype(jnp.int32)
    out16 = pl.kernel(
        _oracle_body,
        out_type=jax.ShapeDtypeStruct((_L,), jnp.float32),
        scratch_types=[
            pltpu.VMEM((_L,), jnp.int32),
            pltpu.VMEM((_L,), jnp.float32),
        ],
        mesh=plsc.VectorSubcoreMesh(core_axis_name="c", subcore_axis_name="s"),
        compiler_params=pltpu.CompilerParams(needs_layout_passes=False),
    )(lu)
    return out16[:10].reshape(1, 10)


# speedup vs baseline: 1.1337x; 1.1337x over previous
import jax
import jax.numpy as jnp
from jax import lax
from jax.experimental import pallas as pl
from jax.experimental.pallas import tpu as pltpu
from jax.experimental.pallas import tpu_sc as plsc

_L = 16


def _oracle_body(lu_hbm, out_hbm, lu_v, out_v):
    cid = lax.axis_index("c")
    sid = lax.axis_index("s")

    @pl.when(jnp.logical_and(cid == 0, sid == 0))
    def _():
        pltpu.sync_copy(lu_hbm, lu_v.at[pl.ds(0, 4)])
        iota = lax.iota(jnp.int32, _L)
        zeros = jnp.zeros((_L,), jnp.int32)
        y_tl = plsc.load_gather(lu_v, [zeros])
        sel = jnp.where(y_tl <= 3, 1, jnp.where(y_tl <= 6, 2, 3))
        fo = plsc.load_gather(lu_v, [sel])
        out_v[...] = jnp.where(iota == fo, 1.0, 0.0).astype(jnp.float32)
        pltpu.sync_copy(out_v.at[pl.ds(0, 10)], out_hbm.at[0])


def kernel(input_ids, look_up):
    del input_ids
    lu = look_up.astype(jnp.int32)
    return pl.kernel(
        _oracle_body,
        out_type=jax.ShapeDtypeStruct((1, 10), jnp.float32),
        scratch_types=[
            pltpu.VMEM((_L,), jnp.int32),
            pltpu.VMEM((_L,), jnp.float32),
        ],
        mesh=plsc.VectorSubcoreMesh(
            core_axis_name="c", subcore_axis_name="s", num_cores=1),
        compiler_params=pltpu.CompilerParams(needs_layout_passes=False),
    )(lu)
